# baseline (speedup 1.0000x reference)
import jax
import jax.numpy as jnp
from jax.experimental import pallas as pl

_NC = 50000
_NP = 100000
_NN = 50000


def _classifier_body(x_ref, W1_ref, b1_ref, W2_ref, b2_ref, W3_ref, b3_ref,
                     out_ref):
    mc = jnp.sum(x_ref[0:1, :]) * (1.0 / _NC)
    mp0 = jnp.sum(x_ref[1:2, :]) * (1.0 / _NP)
    mp1 = jnp.sum(x_ref[2:3, :]) * (1.0 / _NP)
    mn = jnp.sum(x_ref[3:4, :]) * (1.0 / _NN)

    def _r(v):
        return v.astype(jnp.bfloat16).astype(jnp.float32)

    W1 = _r(W1_ref[...])
    h = (_r(mc) * W1[0:1, :] + _r(mp0) * W1[1:2, :]
         + _r(mp1) * W1[2:3, :] + _r(mn) * W1[3:4, :]) + b1_ref[...]
    h = jnp.maximum(h, 0.0)
    h = jnp.dot(_r(h), _r(W2_ref[...]),
                preferred_element_type=jnp.float32) + b2_ref[...]
    h = jnp.maximum(h, 0.0)
    out_ref[...] = (jnp.dot(_r(h), _r(W3_ref[...]),
                            preferred_element_type=jnp.float32)
                    + b3_ref[...])


def kernel(x_component, x_port, x_net,
           edge_cp_src, edge_cp_dst, edge_pn_src, edge_pn_dst,
           W_cp1, b_cp1, W_pn1, b_pn1, W_pn2, b_pn2,
           W_l1, b_l1, W_l2, b_l2, W_l3, b_l3):
    one = b_l1[0] * 0.0 + 1.0
    xc = jnp.pad(x_component.T, ((0, 0), (0, _NP - _NC)))
    xn = jnp.pad(x_net.T, ((0, 0), (0, _NP - _NN)))
    feats = jnp.concatenate([xc, x_port.T, xn], axis=0) * one
    out = pl.pallas_call(
        _classifier_body,
        out_shape=jax.ShapeDtypeStruct((1, 16), jnp.float32),
    )(feats,
      W_l1, b_l1.reshape(1, -1),
      W_l2, b_l2.reshape(1, -1),
      W_l3, b_l3.reshape(1, -1))
    return out


# speedup vs baseline: 1.8748x; 1.8748x over previous
import jax
import jax.numpy as jnp
from jax.experimental import pallas as pl

_NC = 50000
_NP = 100000
_NN = 50000


def _classifier_body(xc_ref, xp_ref, xn_ref,
                     W1_ref, b1_ref, W2_ref, b2_ref, W3_ref, b3_ref,
                     out_ref):
    mc = jnp.sum(xc_ref[...]) * (1.0 / _NC)
    mn = jnp.sum(xn_ref[...]) * (1.0 / _NN)
    mp0 = jnp.sum(xp_ref[0:1, :]) * (1.0 / _NP)
    mp1 = jnp.sum(xp_ref[1:2, :]) * (1.0 / _NP)

    def _r(v):
        return v.astype(jnp.bfloat16).astype(jnp.float32)

    W1 = _r(W1_ref[...])
    h = (_r(mc) * W1[0:1, :] + _r(mp0) * W1[1:2, :]
         + _r(mp1) * W1[2:3, :] + _r(mn) * W1[3:4, :]) + b1_ref[...]
    h = jnp.maximum(h, 0.0)
    h = jnp.dot(_r(h), _r(W2_ref[...]),
                preferred_element_type=jnp.float32) + b2_ref[...]
    h = jnp.maximum(h, 0.0)
    out_ref[...] = (jnp.dot(_r(h), _r(W3_ref[...]),
                            preferred_element_type=jnp.float32)
                    + b3_ref[...])


def kernel(x_component, x_port, x_net,
           edge_cp_src, edge_cp_dst, edge_pn_src, edge_pn_dst,
           W_cp1, b_cp1, W_pn1, b_pn1, W_pn2, b_pn2,
           W_l1, b_l1, W_l2, b_l2, W_l3, b_l3):
    xc = x_component.T
    xp = x_port.T
    xn = x_net.T
    out = pl.pallas_call(
        _classifier_body,
        out_shape=jax.ShapeDtypeStruct((1, 16), jnp.float32),
    )(xc, xp, xn,
      W_l1, b_l1.reshape(1, -1),
      W_l2, b_l2.reshape(1, -1),
      W_l3, b_l3.reshape(1, -1))
    return out
